# baseline (speedup 1.0000x reference)
import jax
import jax.numpy as jnp
import numpy as np
from jax import lax
from jax.experimental import pallas as pl
from jax.experimental.pallas import tpu as pltpu
from jax.experimental.pallas import tpu_sc as plsc

B = 4096
D_NUM = 100
N_CAT = 26
CAT_SIZE = 1000
D_TOKEN = 128
N_TOK = D_NUM + N_CAT
R = B * N_CAT

NC = 2
NS = 16
NW = NC * NS
PER_W = R // NW
CHUNK = 128
N_CHUNKS = PER_W // CHUNK

_OFFSETS = np.cumsum([0] + [CAT_SIZE] * (N_CAT - 1)).astype(np.int32)
_OFF_PATTERN = np.tile(_OFFSETS, PER_W // N_CAT).reshape(N_CHUNKS, CHUNK)

_P = np.arange(R, dtype=np.int64)
_DST = (_P // N_CAT) * N_TOK + D_NUM + (_P % N_CAT)
_DST_PATTERN = _DST.astype(np.int32).reshape(NW, N_CHUNKS, CHUNK)


def _sc_insert_body(xcat_hbm, off_hbm, dst_hbm, table_hbm, out_hbm,
                    idx_v, off_v, dst_v, buf0, buf1, sem0, sem1):
    w = lax.axis_index("c") * NS + lax.axis_index("s")

    pltpu.sync_copy(xcat_hbm.at[w], idx_v)
    pltpu.sync_copy(off_hbm, off_v)
    pltpu.sync_copy(dst_hbm.at[w], dst_v)

    def add_offsets(r, carry):
        for i in range(CHUNK // 16):
            s = pl.ds(i * 16, 16)
            idx_v[r, s] = idx_v[r, s] + off_v[r, s]
        return carry

    lax.fori_loop(0, N_CHUNKS, add_offsets, 0)

    bufs = (buf0, buf1)
    sems = (sem0, sem1)
    copies = [None, None]
    copies[0] = pltpu.async_copy(table_hbm.at[idx_v.at[0]], bufs[0], sems[0])
    for c in range(N_CHUNKS):
        if c + 1 < N_CHUNKS:
            copies[(c + 1) % 2] = pltpu.async_copy(
                table_hbm.at[idx_v.at[c + 1]], bufs[(c + 1) % 2], sems[(c + 1) % 2])
        copies[c % 2].wait()
        pltpu.sync_copy(bufs[c % 2], out_hbm.at[dst_v.at[c]])


@jax.jit
def _sc_insert(xcat3d, off2d, dst3d, cat_table, out_ref):
    mesh = plsc.VectorSubcoreMesh(
        core_axis_name="c", subcore_axis_name="s", num_cores=NC, num_subcores=NS)
    return pl.kernel(
        _sc_insert_body,
        out_type=(),
        mesh=mesh,
        scratch_types=[
            pltpu.VMEM((N_CHUNKS, CHUNK), jnp.int32),
            pltpu.VMEM((N_CHUNKS, CHUNK), jnp.int32),
            pltpu.VMEM((N_CHUNKS, CHUNK), jnp.int32),
            pltpu.VMEM((CHUNK, D_TOKEN), jnp.float32),
            pltpu.VMEM((CHUNK, D_TOKEN), jnp.float32),
            pltpu.SemaphoreType.DMA,
            pltpu.SemaphoreType.DMA,
        ],
    )(xcat3d, off2d, dst3d, cat_table, out_ref)


BB = 128


def _numeric_body(x_ref, w_ref, out_ref):
    out_ref[:, :D_NUM, :] = x_ref[...][:, :, None] * w_ref[...][None, :, :]


@jax.jit
def _tc_numeric(x_num, weight):
    return pl.pallas_call(
        _numeric_body,
        grid=(B // BB,),
        in_specs=[
            pl.BlockSpec((BB, D_NUM), lambda i: (i, 0)),
            pl.BlockSpec((D_NUM, D_TOKEN), lambda i: (0, 0)),
        ],
        out_specs=pl.BlockSpec((BB, N_TOK, D_TOKEN), lambda i: (i, 0, 0)),
        out_shape=jax.ShapeDtypeStruct((B, N_TOK, D_TOKEN), jnp.float32),
    )(x_num, weight)


@jax.jit
def _combined(x_num, xcat3d, off2d, dst3d, weight, cat_table):
    out0 = _tc_numeric(x_num, weight)
    ref = jax.new_ref(out0.reshape(B * N_TOK, D_TOKEN))
    _sc_insert(xcat3d, off2d, dst3d, cat_table, ref)
    return ref[...].reshape(B, N_TOK, D_TOKEN)


def kernel(x_num, x_cat, weight, cat_table):
    xcat3d = x_cat.reshape(NW, N_CHUNKS, CHUNK)
    off2d = jnp.asarray(_OFF_PATTERN)
    dst3d = jnp.asarray(_DST_PATTERN)
    return _combined(x_num, xcat3d, off2d, dst3d, weight, cat_table)


# speedup vs baseline: 1.3224x; 1.3224x over previous
import jax
import jax.numpy as jnp
import numpy as np
from jax import lax
from jax.experimental import pallas as pl
from jax.experimental.pallas import tpu as pltpu
from jax.experimental.pallas import tpu_sc as plsc

B = 4096
D_NUM = 100
N_CAT = 26
CAT_SIZE = 1000
D_TOKEN = 128
N_TOK = D_NUM + N_CAT
R = B * N_CAT

NC = 2
NS = 16
NW = NC * NS
PER_W = R // NW
CHUNK = 128
N_CHUNKS = PER_W // CHUNK

_OFFSETS = np.cumsum([0] + [CAT_SIZE] * (N_CAT - 1)).astype(np.int32)
_OFF_PATTERN = np.tile(_OFFSETS, PER_W // N_CAT).reshape(N_CHUNKS, CHUNK)

_P = np.arange(R, dtype=np.int64)
_DST = (_P // N_CAT) * N_TOK + D_NUM + (_P % N_CAT)
_DST_PATTERN = _DST.astype(np.int32).reshape(NW, N_CHUNKS, CHUNK)


def _sc_insert_body(xcat_hbm, off_hbm, dst_hbm, table_hbm, out_hbm,
                    idx_v, off_v, dst_v, buf0, buf1, sem0, sem1):
    w = lax.axis_index("c") * NS + lax.axis_index("s")

    pltpu.sync_copy(xcat_hbm.at[w], idx_v)
    pltpu.sync_copy(off_hbm, off_v)
    pltpu.sync_copy(dst_hbm.at[w], dst_v)

    def add_offsets(r, carry):
        for i in range(CHUNK // 16):
            s = pl.ds(i * 16, 16)
            idx_v[r, s] = idx_v[r, s] + off_v[r, s]
        return carry

    lax.fori_loop(0, N_CHUNKS, add_offsets, 0)

    bufs = (buf0, buf1)
    sems = (sem0, sem1)
    copies = [None, None]
    copies[0] = pltpu.async_copy(table_hbm.at[idx_v.at[0]], bufs[0], sems[0])
    for c in range(N_CHUNKS):
        if c + 1 < N_CHUNKS:
            copies[(c + 1) % 2] = pltpu.async_copy(
                table_hbm.at[idx_v.at[c + 1]], bufs[(c + 1) % 2], sems[(c + 1) % 2])
        copies[c % 2].wait()
        pltpu.sync_copy(bufs[c % 2], out_hbm.at[dst_v.at[c]])


@jax.jit
def _sc_insert(xcat3d, off2d, dst3d, cat_table, out_ref):
    mesh = plsc.VectorSubcoreMesh(
        core_axis_name="c", subcore_axis_name="s", num_cores=NC, num_subcores=NS)
    return pl.kernel(
        _sc_insert_body,
        out_type=(),
        mesh=mesh,
        scratch_types=[
            pltpu.VMEM((N_CHUNKS, CHUNK), jnp.int32),
            pltpu.VMEM((N_CHUNKS, CHUNK), jnp.int32),
            pltpu.VMEM((N_CHUNKS, CHUNK), jnp.int32),
            pltpu.VMEM((CHUNK, D_TOKEN), jnp.float32),
            pltpu.VMEM((CHUNK, D_TOKEN), jnp.float32),
            pltpu.SemaphoreType.DMA,
            pltpu.SemaphoreType.DMA,
        ],
    )(xcat3d, off2d, dst3d, cat_table, out_ref)


BB = 128


def _numeric_body(x_ref, w_ref, out_ref):
    x3 = x_ref[...][:, :, None]
    w3 = w_ref[...][None, :, :]
    out_ref[...] = (x3 * w3).reshape(BB * N_TOK, D_TOKEN)


@jax.jit
def _tc_numeric(xpad, wpad):
    return pl.pallas_call(
        _numeric_body,
        grid=(B // BB,),
        in_specs=[
            pl.BlockSpec((BB, N_TOK), lambda i: (i, 0)),
            pl.BlockSpec((N_TOK, D_TOKEN), lambda i: (0, 0)),
        ],
        out_specs=pl.BlockSpec((BB * N_TOK, D_TOKEN), lambda i: (i, 0)),
        out_shape=jax.ShapeDtypeStruct((B * N_TOK, D_TOKEN), jnp.float32),
    )(xpad, wpad)


@jax.jit
def _combined(x_num, xcat3d, off2d, dst3d, weight, cat_table):
    xpad = jnp.pad(x_num, ((0, 0), (0, N_CAT)))
    wpad = jnp.pad(weight, ((0, N_CAT), (0, 0)))
    out0 = _tc_numeric(xpad, wpad)
    ref = jax.new_ref(out0)
    _sc_insert(xcat3d, off2d, dst3d, cat_table, ref)
    return ref[...].reshape(B, N_TOK, D_TOKEN)


def kernel(x_num, x_cat, weight, cat_table):
    xcat3d = x_cat.reshape(NW, N_CHUNKS, CHUNK)
    off2d = jnp.asarray(_OFF_PATTERN)
    dst3d = jnp.asarray(_DST_PATTERN)
    return _combined(x_num, xcat3d, off2d, dst3d, weight, cat_table)
